# baseline (speedup 1.0000x reference)
import functools

import jax
import jax.numpy as jnp
from jax import lax
from jax.experimental import pallas as pl
from jax.experimental.pallas import tpu as pltpu
from jax.experimental.pallas import tpu_sc as plsc


_NC, _NS = 2, 16
_CHUNK = 32
_NBUF = 3


def kernel(x, pos_embed):
    b, s, d = x.shape
    nw = _NC * _NS
    rows = s // nw
    n = _CHUNK
    mesh = plsc.VectorSubcoreMesh(core_axis_name="c", subcore_axis_name="s")

    @functools.partial(
        pl.kernel,
        mesh=mesh,
        out_type=jax.ShapeDtypeStruct((b, s, 2 * d), x.dtype),
        scratch_types=(
            [pltpu.VMEM((n, d), jnp.float32)] * _NBUF
            + [pltpu.SemaphoreType.DMA] * (2 * _NBUF)
        ),
    )
    def k(x_hbm, pe_hbm, out_hbm, *scratch):
        bufs = scratch[:_NBUF]
        sin = scratch[_NBUF : 2 * _NBUF]
        sout = scratch[2 * _NBUF :]
        wid = lax.axis_index("s") * _NC + lax.axis_index("c")
        s0 = wid * rows

        work = []
        for bi in range(b):
            for c in range(rows // n):
                work.append(
                    (
                        lambda bi=bi, c=c: x_hbm.at[bi, pl.ds(s0 + c * n, n), :],
                        [
                            lambda bi=bi, c=c: out_hbm.at[
                                bi, pl.ds(s0 + c * n, n), pl.ds(0, d)
                            ]
                        ],
                    )
                )
        for c in range(rows // n):
            work.append(
                (
                    lambda c=c: pe_hbm.at[pl.ds(s0 + c * n, n), :],
                    [
                        lambda bi=bi, c=c: out_hbm.at[
                            bi, pl.ds(s0 + c * n, n), pl.ds(d, d)
                        ]
                        for bi in range(b)
                    ],
                )
            )

        t = len(work)
        gat = [None] * _NBUF
        pend = [[] for _ in range(_NBUF)]

        def issue_gather(i):
            slot = i % _NBUF
            for h in pend[slot]:
                h.wait()
            pend[slot] = []
            gat[slot] = pltpu.async_copy(work[i][0](), bufs[slot], sin[slot])

        issue_gather(0)
        issue_gather(1)
        for i in range(t):
            slot = i % _NBUF
            gat[slot].wait()
            pend[slot] = [
                pltpu.async_copy(bufs[slot], dst(), sout[slot])
                for dst in work[i][1]
            ]
            if i + 2 < t:
                issue_gather(i + 2)
        for slot in range(_NBUF):
            for h in pend[slot]:
                h.wait()

    return k(x, pos_embed)
